# baseline (speedup 1.0000x reference)
import functools

import jax
import jax.numpy as jnp
from jax import lax
from jax.experimental import pallas as pl
from jax.experimental.pallas import tpu as pltpu
from jax.experimental.pallas import tpu_sc as plsc

VOCAB = 1000000
HIDDEN = 64
BATCH = 16384
FIELDS = 26

_B = BATCH * FIELDS
_NW = 32
_B_PER_W = _B // _NW
_CHUNK = 512
_NCHUNK = _B_PER_W // _CHUNK


@functools.partial(
    pl.kernel,
    mesh=plsc.VectorSubcoreMesh(core_axis_name="c", subcore_axis_name="s"),
    out_type=jax.ShapeDtypeStruct((_B, HIDDEN), jnp.float32),
    scratch_types=[
        pltpu.VMEM((_CHUNK,), jnp.int32),
        pltpu.VMEM((_CHUNK, HIDDEN), jnp.float32),
        pltpu.SemaphoreType.DMA,
    ],
    compiler_params=pltpu.CompilerParams(use_tc_tiling_on_sc=False),
)
def _emb_lookup(idx_hbm, table_hbm, out_hbm, idx_v, rows_v, sem):
    wid = lax.axis_index("s") * 2 + lax.axis_index("c")
    base = wid * _B_PER_W

    def body(g, carry):
        off = base + g * _CHUNK
        pltpu.sync_copy(idx_hbm.at[pl.ds(off, _CHUNK)], idx_v)
        pltpu.async_copy(table_hbm.at[idx_v], rows_v, sem).wait()
        pltpu.sync_copy(rows_v, out_hbm.at[pl.ds(off, _CHUNK)])
        return carry

    lax.fori_loop(0, _NCHUNK, body, 0)


def kernel(x, embedding):
    flat = x.reshape(_B)
    out = _emb_lookup(flat, embedding)
    return out.reshape(BATCH, FIELDS, HIDDEN)


# speedup vs baseline: 1.0326x; 1.0326x over previous
import functools

import jax
import jax.numpy as jnp
from jax import lax
from jax.experimental import pallas as pl
from jax.experimental.pallas import tpu as pltpu
from jax.experimental.pallas import tpu_sc as plsc

VOCAB = 1000000
HIDDEN = 64
BATCH = 16384
FIELDS = 26

_B = BATCH * FIELDS
_NW = 32
_B_PER_W = _B // _NW
_CHUNK = 832
_NCHUNK = _B_PER_W // _CHUNK
_NBUF = 2


@functools.partial(
    pl.kernel,
    mesh=plsc.VectorSubcoreMesh(core_axis_name="c", subcore_axis_name="s"),
    out_type=jax.ShapeDtypeStruct((_B, HIDDEN), jnp.float32),
    scratch_types=[
        pltpu.VMEM((_B_PER_W,), jnp.int32),
        pltpu.VMEM((_NBUF, _CHUNK, HIDDEN), jnp.float32),
        pltpu.SemaphoreType.DMA((_NBUF,)),
        pltpu.SemaphoreType.DMA((_NBUF,)),
    ],
    compiler_params=pltpu.CompilerParams(use_tc_tiling_on_sc=False),
)
def _emb_lookup(idx_hbm, table_hbm, out_hbm, idx_v, rows_v, gsem, osem):
    wid = lax.axis_index("s") * 2 + lax.axis_index("c")
    base = wid * _B_PER_W

    pltpu.sync_copy(idx_hbm.at[pl.ds(base, _B_PER_W)], idx_v)

    gd = [None] * _NCHUNK
    od = [None] * _NCHUNK
    for g in range(_NCHUNK + 1):
        b = g % _NBUF
        if g < _NCHUNK:
            if g >= _NBUF:
                od[g - _NBUF].wait()
            gd[g] = pltpu.async_copy(
                table_hbm.at[idx_v.at[pl.ds(g * _CHUNK, _CHUNK)]],
                rows_v.at[b],
                gsem.at[b],
            )
        if g >= 1:
            p = g - 1
            gd[p].wait()
            od[p] = pltpu.async_copy(
                rows_v.at[p % _NBUF],
                out_hbm.at[pl.ds(base + p * _CHUNK, _CHUNK)],
                osem.at[p % _NBUF],
            )
    od[_NCHUNK - 2].wait()
    od[_NCHUNK - 1].wait()


def kernel(x, embedding):
    flat = x.reshape(_B)
    out = _emb_lookup(flat, embedding)
    return out.reshape(BATCH, FIELDS, HIDDEN)
